# baseline (speedup 1.0000x reference)
import functools

import jax
import jax.numpy as jnp
from jax import lax
from jax.experimental import pallas as pl
from jax.experimental.pallas import tpu as pltpu
from jax.experimental.pallas import tpu_sc as plsc

_SCENE = (256, 256, 32)
_PS = 2
_SX, _SY, _SZ = _SCENE[0] // _PS, _SCENE[1] // _PS, _SCENE[2] // _PS

_C, _H, _W = 128, 96, 320
_HW = _H * _W
_N = _SX * _SY * _SZ

_NWORKERS = 32
_SL = _N // _NWORKERS
_CPW = _C // _NWORKERS
_CH = 512
_NCH = _N // _CH
_YPC = _CH // _SZ
_CPX = _SY // _YPC

_mesh = plsc.VectorSubcoreMesh(core_axis_name="c", subcore_axis_name="s")
_params = pltpu.CompilerParams(needs_layout_passes=False,
                               use_tc_tiling_on_sc=False)


@functools.partial(
    pl.kernel,
    out_type=[
        jax.ShapeDtypeStruct((_N,), jnp.int32),
        jax.ShapeDtypeStruct((_N,), jnp.float32),
    ],
    mesh=_mesh,
    compiler_params=_params,
    scratch_types=[
        pltpu.VMEM((_HW,), jnp.float32),
        pltpu.VMEM((2 * _SL,), jnp.int32),
        pltpu.VMEM((16,), jnp.float32),
        pltpu.VMEM((_SL,), jnp.float32),
        pltpu.VMEM((_SL,), jnp.float32),
        pltpu.VMEM((_SL,), jnp.int32),
        pltpu.VMEM((_SL,), jnp.float32),
    ],
)
def _idx_weight(pix_hbm, rcp_hbm, fov_hbm, pz_hbm, depth_hbm,
                idx_hbm, w_hbm,
                depth_v, pix_v, rcp_v, fov_v, pz_v, idx_v, w_v):
    wid = lax.axis_index("s") * 2 + lax.axis_index("c")
    base = wid * _SL
    pltpu.sync_copy(depth_hbm, depth_v)
    pltpu.sync_copy(pix_hbm.at[pl.ds(2 * base, 2 * _SL)], pix_v)
    pltpu.sync_copy(rcp_hbm, rcp_v)
    pltpu.sync_copy(fov_hbm.at[pl.ds(base, _SL)], fov_v)
    pltpu.sync_copy(pz_hbm.at[pl.ds(base, _SL)], pz_v)

    iota2 = 2 * lax.iota(jnp.int32, 16)
    rcp = rcp_v[...]

    @plsc.parallel_loop(0, _SL // 16, unroll=2)
    def _(j):
        o = j * 16
        p2 = 2 * o + iota2
        x = plsc.load_gather(pix_v, [p2])
        y = plsc.load_gather(pix_v, [p2 + 1])
        di = y * _W + x
        xs_ = ((x.astype(jnp.float32) + 0.5) * rcp).astype(jnp.int32)
        ys_ = ((y.astype(jnp.float32) + 0.5) * rcp).astype(jnp.int32)
        idx_v[pl.ds(o, 16)] = ys_ * _W + xs_
        d = plsc.load_gather(depth_v, [di])
        t = pz_v[pl.ds(o, 16)] - d
        wgt = jnp.exp(t * t * -2.0)
        wgt = jnp.where(d == 0.0, jnp.float32(1.0), wgt)
        w_v[pl.ds(o, 16)] = wgt * fov_v[pl.ds(o, 16)]

    pltpu.sync_copy(idx_v, idx_hbm.at[pl.ds(base, _SL)])
    pltpu.sync_copy(w_v, w_hbm.at[pl.ds(base, _SL)])


@functools.partial(
    pl.kernel,
    out_type=jax.ShapeDtypeStruct((_C, _SX, _SY, _SZ), jnp.float32),
    mesh=_mesh,
    compiler_params=_params,
    scratch_types=[
        pltpu.VMEM((_CPW * _HW,), jnp.float32),
        pltpu.VMEM((2, _CH), jnp.int32),
        pltpu.VMEM((2, _CH), jnp.float32),
        pltpu.VMEM((2, _CPW, 1, _YPC, _SZ), jnp.float32),
        pltpu.SemaphoreType.DMA,
        pltpu.SemaphoreType.DMA,
        pltpu.SemaphoreType.DMA,
        pltpu.SemaphoreType.DMA,
    ],
)
def _gather_scale(src_hbm, idx_hbm, w_hbm, out_hbm,
                  rows_v, idx2, w2, out2, sin0, sin1, sout0, sout1):
    wid = lax.axis_index("s") * 2 + lax.axis_index("c")
    c0 = wid * _CPW
    sins = (sin0, sin1)
    souts = (sout0, sout1)

    def start_in(k, b):
        n0 = k * _CH
        pltpu.async_copy(idx_hbm.at[pl.ds(n0, _CH)], idx2.at[b], sins[b])
        pltpu.async_copy(w_hbm.at[pl.ds(n0, _CH)], w2.at[b], sins[b])

    def wait_in(k, b):
        n0 = k * _CH
        pltpu.make_async_copy(idx_hbm.at[pl.ds(n0, _CH)], idx2.at[b], sins[b]).wait()
        pltpu.make_async_copy(w_hbm.at[pl.ds(n0, _CH)], w2.at[b], sins[b]).wait()

    def out_copy(k, b):
        q = k // _CPX
        r = (k % _CPX) * _YPC
        return pltpu.make_async_copy(
            out2.at[b],
            out_hbm.at[pl.ds(c0, _CPW), pl.ds(q, 1), pl.ds(r, _YPC), pl.ds(0, _SZ)],
            souts[b])

    start_in(0, 0)
    start_in(1, 1)
    for c in range(_CPW):
        pltpu.sync_copy(src_hbm.at[c0 + c], rows_v.at[pl.ds(c * _HW, _HW)])

    def step(i, carry):
        for b in range(2):
            k = 2 * i + b
            wait_in(k, b)

            @pl.when(i >= 1)
            def _():
                out_copy(k - 2, b).wait()

            @plsc.parallel_loop(0, _CH // 16, unroll=4)
            def _(j):
                o = j * 16
                iv = idx2[b, pl.ds(o, 16)]
                wv = w2[b, pl.ds(o, 16)]
                for c in range(_CPW):
                    g = plsc.load_gather(rows_v, [iv + c * _HW])
                    out2[b, c, 0, j, :] = g * wv

            out_copy(k, b).start()

            @pl.when(i < _NCH // 2 - 1)
            def _():
                start_in(k + 2, b)
        return carry

    lax.fori_loop(0, _NCH // 2, step, 0)
    out_copy(_NCH - 2, 0).wait()
    out_copy(_NCH - 1, 1).wait()


def kernel(x2d, projected_pix, scale_2d, fov_mask, pix_z, depth_img):
    c, h, w = x2d.shape
    pix_flat = projected_pix.reshape(-1)
    rcp_vec = jnp.full((16,), 1.0, jnp.float32) / jnp.float32(scale_2d)
    fov_f = fov_mask.astype(jnp.float32)
    pz = pix_z.reshape(-1)
    depth_flat = depth_img.reshape(-1)
    src = x2d.reshape(c, h * w)

    idx, wgt = _idx_weight(pix_flat, rcp_vec, fov_f, pz, depth_flat)
    return _gather_scale(src, idx, wgt)


# speedup vs baseline: 1.1313x; 1.1313x over previous
import functools

import jax
import jax.numpy as jnp
from jax import lax
from jax.experimental import pallas as pl
from jax.experimental.pallas import tpu as pltpu
from jax.experimental.pallas import tpu_sc as plsc

_SCENE = (256, 256, 32)
_PS = 2
_SX, _SY, _SZ = _SCENE[0] // _PS, _SCENE[1] // _PS, _SCENE[2] // _PS

_C, _H, _W = 128, 96, 320
_HW = _H * _W
_N = _SX * _SY * _SZ

_NWORKERS = 32
_SL = _N // _NWORKERS
_CPW = _C // _NWORKERS
_CH = 512
_NCH = _N // _CH
_YPC = _CH // _SZ
_CPX = _SY // _YPC

_mesh = plsc.VectorSubcoreMesh(core_axis_name="c", subcore_axis_name="s")
_params = pltpu.CompilerParams(needs_layout_passes=False,
                               use_tc_tiling_on_sc=False)


@functools.partial(
    pl.kernel,
    out_type=[
        jax.ShapeDtypeStruct((_N,), jnp.int32),
        jax.ShapeDtypeStruct((_N,), jnp.float32),
    ],
    mesh=_mesh,
    compiler_params=_params,
    scratch_types=[
        pltpu.VMEM((_HW,), jnp.float32),
        pltpu.VMEM((_SL,), jnp.int32),
        pltpu.VMEM((_SL,), jnp.int32),
        pltpu.VMEM((16,), jnp.float32),
        pltpu.VMEM((_SL,), jnp.float32),
        pltpu.VMEM((_SL,), jnp.float32),
        pltpu.VMEM((_SL,), jnp.int32),
        pltpu.VMEM((_SL,), jnp.float32),
    ],
)
def _idx_weight(xs_hbm, ys_hbm, rcp_hbm, fov_hbm, pz_hbm, depth_hbm,
                idx_hbm, w_hbm,
                depth_v, xs_v, ys_v, rcp_v, fov_v, pz_v, idx_v, w_v):
    wid = lax.axis_index("s") * 2 + lax.axis_index("c")
    base = wid * _SL
    pltpu.sync_copy(depth_hbm, depth_v)
    pltpu.sync_copy(xs_hbm.at[pl.ds(base, _SL)], xs_v)
    pltpu.sync_copy(ys_hbm.at[pl.ds(base, _SL)], ys_v)
    pltpu.sync_copy(rcp_hbm, rcp_v)
    pltpu.sync_copy(fov_hbm.at[pl.ds(base, _SL)], fov_v)
    pltpu.sync_copy(pz_hbm.at[pl.ds(base, _SL)], pz_v)

    rcp = rcp_v[...]

    @plsc.parallel_loop(0, _SL // 16, unroll=2)
    def _(j):
        o = j * 16
        x = xs_v[pl.ds(o, 16)]
        y = ys_v[pl.ds(o, 16)]
        di = y * _W + x
        xs_ = ((x.astype(jnp.float32) + 0.5) * rcp).astype(jnp.int32)
        ys_ = ((y.astype(jnp.float32) + 0.5) * rcp).astype(jnp.int32)
        idx_v[pl.ds(o, 16)] = ys_ * _W + xs_
        d = plsc.load_gather(depth_v, [di])
        t = pz_v[pl.ds(o, 16)] - d
        wgt = jnp.exp(t * t * -2.0)
        wgt = jnp.where(d == 0.0, jnp.float32(1.0), wgt)
        w_v[pl.ds(o, 16)] = wgt * fov_v[pl.ds(o, 16)]

    pltpu.sync_copy(idx_v, idx_hbm.at[pl.ds(base, _SL)])
    pltpu.sync_copy(w_v, w_hbm.at[pl.ds(base, _SL)])


@functools.partial(
    pl.kernel,
    out_type=jax.ShapeDtypeStruct((_C, _SX, _SY, _SZ), jnp.float32),
    mesh=_mesh,
    compiler_params=_params,
    scratch_types=[
        pltpu.VMEM((_CPW * _HW,), jnp.float32),
        pltpu.VMEM((2, _CH), jnp.int32),
        pltpu.VMEM((2, _CH), jnp.float32),
        pltpu.VMEM((2, _CPW, 1, _YPC, _SZ), jnp.float32),
        pltpu.SemaphoreType.DMA,
        pltpu.SemaphoreType.DMA,
        pltpu.SemaphoreType.DMA,
        pltpu.SemaphoreType.DMA,
    ],
)
def _gather_scale(src_hbm, idx_hbm, w_hbm, out_hbm,
                  rows_v, idx2, w2, out2, sin0, sin1, sout0, sout1):
    wid = lax.axis_index("s") * 2 + lax.axis_index("c")
    c0 = wid * _CPW
    sins = (sin0, sin1)
    souts = (sout0, sout1)

    def start_in(k, b):
        n0 = k * _CH
        pltpu.async_copy(idx_hbm.at[pl.ds(n0, _CH)], idx2.at[b], sins[b])
        pltpu.async_copy(w_hbm.at[pl.ds(n0, _CH)], w2.at[b], sins[b])

    def wait_in(k, b):
        n0 = k * _CH
        pltpu.make_async_copy(idx_hbm.at[pl.ds(n0, _CH)], idx2.at[b], sins[b]).wait()
        pltpu.make_async_copy(w_hbm.at[pl.ds(n0, _CH)], w2.at[b], sins[b]).wait()

    def out_copy(k, b):
        q = k // _CPX
        r = (k % _CPX) * _YPC
        return pltpu.make_async_copy(
            out2.at[b],
            out_hbm.at[pl.ds(c0, _CPW), pl.ds(q, 1), pl.ds(r, _YPC), pl.ds(0, _SZ)],
            souts[b])

    start_in(0, 0)
    start_in(1, 1)
    for c in range(_CPW):
        pltpu.sync_copy(src_hbm.at[c0 + c], rows_v.at[pl.ds(c * _HW, _HW)])

    def step(i, carry):
        for b in range(2):
            k = 2 * i + b
            wait_in(k, b)

            @pl.when(i >= 1)
            def _():
                out_copy(k - 2, b).wait()

            @plsc.parallel_loop(0, _CH // 16, unroll=4)
            def _(j):
                o = j * 16
                iv = idx2[b, pl.ds(o, 16)]
                wv = w2[b, pl.ds(o, 16)]
                for c in range(_CPW):
                    g = plsc.load_gather(rows_v, [iv + c * _HW])
                    out2[b, c, 0, j, :] = g * wv

            out_copy(k, b).start()

            @pl.when(i < _NCH // 2 - 1)
            def _():
                start_in(k + 2, b)
        return carry

    lax.fori_loop(0, _NCH // 2, step, 0)
    out_copy(_NCH - 2, 0).wait()
    out_copy(_NCH - 1, 1).wait()


def kernel(x2d, projected_pix, scale_2d, fov_mask, pix_z, depth_img):
    c, h, w = x2d.shape
    xs = projected_pix[:, 0]
    ys = projected_pix[:, 1]
    rcp_vec = jnp.full((16,), 1.0, jnp.float32) / jnp.float32(scale_2d)
    fov_f = fov_mask.astype(jnp.float32)
    pz = pix_z.reshape(-1)
    depth_flat = depth_img.reshape(-1)
    src = x2d.reshape(c, h * w)

    idx, wgt = _idx_weight(xs, ys, rcp_vec, fov_f, pz, depth_flat)
    return _gather_scale(src, idx, wgt)


# speedup vs baseline: 3.7906x; 3.3507x over previous
import functools

import jax
import jax.numpy as jnp
from jax import lax
from jax.experimental import pallas as pl
from jax.experimental.pallas import tpu as pltpu
from jax.experimental.pallas import tpu_sc as plsc

_SCENE = (256, 256, 32)
_PS = 2
_SX, _SY, _SZ = _SCENE[0] // _PS, _SCENE[1] // _PS, _SCENE[2] // _PS

_C, _H, _W = 128, 96, 320
_HW = _H * _W
_N = _SX * _SY * _SZ
_XS = _SY * _SZ

_NWORKERS = 32
_SL = _N // _NWORKERS
_CPW = _C // _NWORKERS
_CH = 512
_NCH = _N // _CH
_ZPC = _CH // _SY
_CPX = _XS // _CH

_mesh = plsc.VectorSubcoreMesh(core_axis_name="c", subcore_axis_name="s")
_params = pltpu.CompilerParams(needs_layout_passes=False,
                               use_tc_tiling_on_sc=False)


@functools.partial(
    pl.kernel,
    out_type=[
        jax.ShapeDtypeStruct((_N,), jnp.int32),
        jax.ShapeDtypeStruct((_N,), jnp.float32),
    ],
    mesh=_mesh,
    compiler_params=_params,
    scratch_types=[
        pltpu.VMEM((_HW,), jnp.float32),
        pltpu.VMEM((_SL,), jnp.int32),
        pltpu.VMEM((_SL,), jnp.int32),
        pltpu.VMEM((16,), jnp.float32),
        pltpu.VMEM((_SL,), jnp.float32),
        pltpu.VMEM((_SL,), jnp.float32),
        pltpu.VMEM((_SL,), jnp.int32),
        pltpu.VMEM((_SL,), jnp.float32),
    ],
)
def _idx_weight(xs_hbm, ys_hbm, rcp_hbm, fov_hbm, pz_hbm, depth_hbm,
                idx_hbm, w_hbm,
                depth_v, xs_v, ys_v, rcp_v, fov_v, pz_v, idx_v, w_v):
    wid = lax.axis_index("s") * 2 + lax.axis_index("c")
    base = wid * _SL
    pltpu.sync_copy(depth_hbm, depth_v)
    pltpu.sync_copy(xs_hbm.at[pl.ds(base, _SL)], xs_v)
    pltpu.sync_copy(ys_hbm.at[pl.ds(base, _SL)], ys_v)
    pltpu.sync_copy(rcp_hbm, rcp_v)
    pltpu.sync_copy(fov_hbm.at[pl.ds(base, _SL)], fov_v)
    pltpu.sync_copy(pz_hbm.at[pl.ds(base, _SL)], pz_v)

    rcp = rcp_v[...]
    iota16 = lax.iota(jnp.int32, 16)

    @plsc.parallel_loop(0, _SL // 16, unroll=2)
    def _(j):
        m = j * 16
        nv = (j // 128) * _XS + ((j % 8) * 16 + iota16) * _SZ + (j // 8) % 16
        x = plsc.load_gather(xs_v, [nv])
        y = plsc.load_gather(ys_v, [nv])
        di = y * _W + x
        xs_ = ((x.astype(jnp.float32) + 0.5) * rcp).astype(jnp.int32)
        ys_ = ((y.astype(jnp.float32) + 0.5) * rcp).astype(jnp.int32)
        idx_v[pl.ds(m, 16)] = ys_ * _W + xs_
        d = plsc.load_gather(depth_v, [di])
        t = plsc.load_gather(pz_v, [nv]) - d
        wgt = jnp.exp(t * t * -2.0)
        wgt = jnp.where(d == 0.0, jnp.float32(1.0), wgt)
        w_v[pl.ds(m, 16)] = wgt * plsc.load_gather(fov_v, [nv])

    pltpu.sync_copy(idx_v, idx_hbm.at[pl.ds(base, _SL)])
    pltpu.sync_copy(w_v, w_hbm.at[pl.ds(base, _SL)])


@functools.partial(
    pl.kernel,
    out_type=jax.ShapeDtypeStruct((_C, _SX, _SZ, _SY), jnp.float32),
    mesh=_mesh,
    compiler_params=_params,
    scratch_types=[
        pltpu.VMEM((_CPW * _HW,), jnp.float32),
        pltpu.VMEM((2, _CH), jnp.int32),
        pltpu.VMEM((2, _CH), jnp.float32),
        pltpu.VMEM((2, _CPW, 1, _ZPC, _SY), jnp.float32),
        pltpu.SemaphoreType.DMA,
        pltpu.SemaphoreType.DMA,
        pltpu.SemaphoreType.DMA,
        pltpu.SemaphoreType.DMA,
    ],
)
def _gather_scale(src_hbm, idx_hbm, w_hbm, out_hbm,
                  rows_v, idx2, w2, out2, sin0, sin1, sout0, sout1):
    wid = lax.axis_index("s") * 2 + lax.axis_index("c")
    c0 = wid * _CPW
    sins = (sin0, sin1)
    souts = (sout0, sout1)

    def start_in(k, b):
        n0 = k * _CH
        pltpu.async_copy(idx_hbm.at[pl.ds(n0, _CH)], idx2.at[b], sins[b])
        pltpu.async_copy(w_hbm.at[pl.ds(n0, _CH)], w2.at[b], sins[b])

    def wait_in(k, b):
        n0 = k * _CH
        pltpu.make_async_copy(idx_hbm.at[pl.ds(n0, _CH)], idx2.at[b], sins[b]).wait()
        pltpu.make_async_copy(w_hbm.at[pl.ds(n0, _CH)], w2.at[b], sins[b]).wait()

    def out_copy(k, b):
        q = k // _CPX
        zb = (k % _CPX) * _ZPC
        return pltpu.make_async_copy(
            out2.at[b],
            out_hbm.at[pl.ds(c0, _CPW), pl.ds(q, 1), pl.ds(zb, _ZPC),
                       pl.ds(0, _SY)],
            souts[b])

    start_in(0, 0)
    start_in(1, 1)
    for c in range(_CPW):
        pltpu.sync_copy(src_hbm.at[c0 + c], rows_v.at[pl.ds(c * _HW, _HW)])

    def step(i, carry):
        for b in range(2):
            k = 2 * i + b
            wait_in(k, b)

            @pl.when(i >= 1)
            def _():
                out_copy(k - 2, b).wait()

            @plsc.parallel_loop(0, _CH // 16, unroll=4)
            def _(j):
                o = j * 16
                iv = idx2[b, pl.ds(o, 16)]
                wv = w2[b, pl.ds(o, 16)]
                zl = j // 8
                y0 = (j % 8) * 16
                for c in range(_CPW):
                    g = plsc.load_gather(rows_v, [iv + c * _HW])
                    out2[b, c, 0, zl, pl.ds(y0, 16)] = g * wv

            out_copy(k, b).start()

            @pl.when(i < _NCH // 2 - 1)
            def _():
                start_in(k + 2, b)
        return carry

    lax.fori_loop(0, _NCH // 2, step, 0)
    out_copy(_NCH - 2, 0).wait()
    out_copy(_NCH - 1, 1).wait()


def kernel(x2d, projected_pix, scale_2d, fov_mask, pix_z, depth_img):
    c, h, w = x2d.shape
    xs = projected_pix[:, 0]
    ys = projected_pix[:, 1]
    rcp_vec = jnp.full((16,), 1.0, jnp.float32) / jnp.float32(scale_2d)
    fov_f = fov_mask.astype(jnp.float32)
    pz = pix_z.reshape(-1)
    depth_flat = depth_img.reshape(-1)
    src = x2d.reshape(c, h * w)

    idx, wgt = _idx_weight(xs, ys, rcp_vec, fov_f, pz, depth_flat)
    out_czy = _gather_scale(src, idx, wgt)
    return jnp.transpose(out_czy, (0, 1, 3, 2))


# speedup vs baseline: 4.1240x; 1.0880x over previous
import functools

import jax
import jax.numpy as jnp
from jax import lax
from jax.experimental import pallas as pl
from jax.experimental.pallas import tpu as pltpu
from jax.experimental.pallas import tpu_sc as plsc

_SCENE = (256, 256, 32)
_PS = 2
_SX, _SY, _SZ = _SCENE[0] // _PS, _SCENE[1] // _PS, _SCENE[2] // _PS

_C, _H, _W = 128, 96, 320
_HW = _H * _W
_N = _SX * _SY * _SZ
_XS = _SY * _SZ

_NWORKERS = 32
_SL = _N // _NWORKERS
_CPW = _C // _NWORKERS
_CH = 512
_NCH = _N // _CH

_mesh = plsc.VectorSubcoreMesh(core_axis_name="c", subcore_axis_name="s")
_params = pltpu.CompilerParams(needs_layout_passes=False,
                               use_tc_tiling_on_sc=False)


@functools.partial(
    pl.kernel,
    out_type=jax.ShapeDtypeStruct((_N,), jnp.int32),
    mesh=_mesh,
    compiler_params=_params,
    scratch_types=[
        pltpu.VMEM((_HW,), jnp.float32),
        pltpu.VMEM((_SL,), jnp.int32),
        pltpu.VMEM((_SL,), jnp.int32),
        pltpu.VMEM((16,), jnp.float32),
        pltpu.VMEM((_SL,), jnp.float32),
        pltpu.VMEM((_SL,), jnp.float32),
        pltpu.VMEM((_SL,), jnp.int32),
    ],
)
def _idx_weight(xs_hbm, ys_hbm, rcp_hbm, fov_hbm, pz_hbm, depth_hbm,
                pk_hbm,
                depth_v, xs_v, ys_v, rcp_v, fov_v, pz_v, pk_v):
    wid = lax.axis_index("s") * 2 + lax.axis_index("c")
    base = wid * _SL
    pltpu.sync_copy(depth_hbm, depth_v)
    pltpu.sync_copy(xs_hbm.at[pl.ds(base, _SL)], xs_v)
    pltpu.sync_copy(ys_hbm.at[pl.ds(base, _SL)], ys_v)
    pltpu.sync_copy(rcp_hbm, rcp_v)
    pltpu.sync_copy(fov_hbm.at[pl.ds(base, _SL)], fov_v)
    pltpu.sync_copy(pz_hbm.at[pl.ds(base, _SL)], pz_v)

    rcp = rcp_v[...]
    iota16 = lax.iota(jnp.int32, 16)

    @plsc.parallel_loop(0, _SL // 16, unroll=2)
    def _(j):
        m = j * 16
        nv = (j // 128) * _XS + ((j % 8) * 16 + iota16) * _SZ + (j // 8) % 16
        x = plsc.load_gather(xs_v, [nv])
        y = plsc.load_gather(ys_v, [nv])
        di = y * _W + x
        xs_ = ((x.astype(jnp.float32) + 0.5) * rcp).astype(jnp.int32)
        ys_ = ((y.astype(jnp.float32) + 0.5) * rcp).astype(jnp.int32)
        d = plsc.load_gather(depth_v, [di])
        t = plsc.load_gather(pz_v, [nv]) - d
        wgt = jnp.exp(t * t * -2.0)
        wgt = jnp.where(d == 0.0, jnp.float32(1.0), wgt)
        wgt = wgt * plsc.load_gather(fov_v, [nv])
        wb = plsc.bitcast(wgt, jnp.int32)
        wb = (wb + 0x8000) & jnp.int32(-65536)
        pk_v[pl.ds(m, 16)] = wb | (ys_ * _W + xs_)

    pltpu.sync_copy(pk_v, pk_hbm.at[pl.ds(base, _SL)])


@functools.partial(
    pl.kernel,
    out_type=jax.ShapeDtypeStruct((_C, _N), jnp.float32),
    mesh=_mesh,
    compiler_params=_params,
    scratch_types=[
        pltpu.VMEM((_CPW * _HW,), jnp.float32),
        pltpu.VMEM((2, _CH), jnp.int32),
        pltpu.VMEM((2, _CPW, _CH), jnp.float32),
        pltpu.SemaphoreType.DMA,
        pltpu.SemaphoreType.DMA,
        pltpu.SemaphoreType.DMA,
        pltpu.SemaphoreType.DMA,
    ],
)
def _gather_scale(src_hbm, pk_hbm, out_hbm,
                  rows_v, pk2, out2, sin0, sin1, sout0, sout1):
    wid = lax.axis_index("s") * 2 + lax.axis_index("c")
    c0 = wid * _CPW
    sins = (sin0, sin1)
    souts = (sout0, sout1)

    def start_in(k, b):
        pltpu.async_copy(pk_hbm.at[pl.ds(k * _CH, _CH)], pk2.at[b], sins[b])

    def wait_in(k, b):
        pltpu.make_async_copy(pk_hbm.at[pl.ds(k * _CH, _CH)], pk2.at[b],
                              sins[b]).wait()

    def out_copy(k, b):
        return pltpu.make_async_copy(
            out2.at[b],
            out_hbm.at[pl.ds(c0, _CPW), pl.ds(k * _CH, _CH)],
            souts[b])

    start_in(0, 0)
    start_in(1, 1)
    for c in range(_CPW):
        pltpu.sync_copy(src_hbm.at[c0 + c], rows_v.at[pl.ds(c * _HW, _HW)])

    def step(i, carry):
        for b in range(2):
            k = 2 * i + b
            wait_in(k, b)

            @pl.when(i >= 1)
            def _():
                out_copy(k - 2, b).wait()

            @plsc.parallel_loop(0, _CH // 16, unroll=8)
            def _(j):
                o = j * 16
                pk = pk2[b, pl.ds(o, 16)]
                iv = pk & 0xFFFF
                wv = plsc.bitcast(pk & jnp.int32(-65536), jnp.float32)
                for c in range(_CPW):
                    g = plsc.load_gather(rows_v, [iv + c * _HW])
                    out2[b, c, pl.ds(o, 16)] = g * wv

            out_copy(k, b).start()

            @pl.when(i < _NCH // 2 - 1)
            def _():
                start_in(k + 2, b)
        return carry

    lax.fori_loop(0, _NCH // 2, step, 0)
    out_copy(_NCH - 2, 0).wait()
    out_copy(_NCH - 1, 1).wait()


def kernel(x2d, projected_pix, scale_2d, fov_mask, pix_z, depth_img):
    c, h, w = x2d.shape
    xs = projected_pix[:, 0]
    ys = projected_pix[:, 1]
    rcp_vec = jnp.full((16,), 1.0, jnp.float32) / jnp.float32(scale_2d)
    fov_f = fov_mask.astype(jnp.float32)
    pz = pix_z.reshape(-1)
    depth_flat = depth_img.reshape(-1)
    src = x2d.reshape(c, h * w)

    pk = _idx_weight(xs, ys, rcp_vec, fov_f, pz, depth_flat)
    out = _gather_scale(src, pk)
    return jnp.transpose(out.reshape(c, _SX, _SZ, _SY), (0, 1, 3, 2))
